# Initial kernel scaffold; baseline (speedup 1.0000x reference)
import jax
import jax.numpy as jnp
from jax.experimental import pallas as pl


def kernel(proj_vars, res_state, wr_rows, wr_cols, wr_vals):
    raise NotImplementedError("write your pallas kernel here")


# speedup vs baseline: 234.4812x; 234.4812x over previous
import jax
import jax.numpy as jnp
from jax import lax
from jax.experimental import pallas as pl
from jax.experimental.pallas import tpu as pltpu
from jax.experimental.pallas import tpu_sc as plsc

RES = 16384
NNZ = 2684354
LEAK = 0.6
BIAS = 1.6

NC = 2
NS = 16
NW = NC * NS
LANES = 16

CHUNK = 4096
VPC = CHUNK // LANES
NFULL = NNZ // CHUNK
TBASE = NFULL * CHUNK
TAIL = NNZ - TBASE
TFULL = TAIL // LANES
TREM = TAIL - TFULL * LANES


def _spmv_body(res_hbm, rows_hbm, cols_hbm, vals_hbm, out_hbm,
               state_v, acc_v, rowbuf, colbuf, valbuf):
    wid = lax.axis_index("s") * NC + lax.axis_index("c")

    pltpu.sync_copy(res_hbm, state_v)
    zeros = jnp.zeros((LANES,), jnp.float32)

    def _z(i, c):
        acc_v[pl.ds(i * LANES, LANES)] = zeros
        return c
    lax.fori_loop(0, RES // LANES, _z, 0)

    def _vreg(off):
        idx = colbuf[pl.ds(off, LANES)]
        r = rowbuf[pl.ds(off, LANES)]
        v = valbuf[pl.ds(off, LANES)]
        g = plsc.load_gather(state_v, [idx])
        plsc.addupdate_scatter(acc_v, [r], v * g)

    def _chunk(k, c):
        base = (wid + k * NW) * CHUNK
        pltpu.sync_copy(rows_hbm.at[pl.ds(base, CHUNK)], rowbuf)
        pltpu.sync_copy(cols_hbm.at[pl.ds(base, CHUNK)], colbuf)
        pltpu.sync_copy(vals_hbm.at[pl.ds(base, CHUNK)], valbuf)

        def _v(j, cc):
            _vreg(j * LANES)
            return cc
        lax.fori_loop(0, VPC, _v, 0)
        return c

    nmine = (NFULL - wid + NW - 1) // NW
    lax.fori_loop(0, nmine, _chunk, 0)

    @pl.when(wid == NW - 1)
    def _tail():
        pltpu.sync_copy(rows_hbm.at[pl.ds(TBASE, TAIL)], rowbuf.at[pl.ds(0, TAIL)])
        pltpu.sync_copy(cols_hbm.at[pl.ds(TBASE, TAIL)], colbuf.at[pl.ds(0, TAIL)])
        pltpu.sync_copy(vals_hbm.at[pl.ds(TBASE, TAIL)], valbuf.at[pl.ds(0, TAIL)])

        def _v(j, cc):
            _vreg(j * LANES)
            return cc
        lax.fori_loop(0, TFULL, _v, 0)

        off = TFULL * LANES
        m = lax.iota(jnp.int32, LANES) < TREM
        idx = colbuf[pl.ds(off, LANES)]
        r = rowbuf[pl.ds(off, LANES)]
        v = valbuf[pl.ds(off, LANES)]
        g = plsc.load_gather(state_v, [idx], mask=m)
        plsc.addupdate_scatter(acc_v, [r], v * g, mask=m)

    pltpu.sync_copy(acc_v, out_hbm.at[wid])


_spmv = pl.kernel(
    _spmv_body,
    out_type=jax.ShapeDtypeStruct((NW, RES), jnp.float32),
    mesh=plsc.VectorSubcoreMesh(core_axis_name="c", subcore_axis_name="s"),
    compiler_params=pltpu.CompilerParams(needs_layout_passes=False),
    scratch_types=[
        pltpu.VMEM((RES,), jnp.float32),
        pltpu.VMEM((RES,), jnp.float32),
        pltpu.VMEM((CHUNK,), jnp.int32),
        pltpu.VMEM((CHUNK,), jnp.int32),
        pltpu.VMEM((CHUNK,), jnp.float32),
    ],
)


def _finish_body(part_ref, res_ref, proj_ref, out_ref):
    s = jnp.sum(part_ref[...], axis=0)
    out_ref[...] = ((1.0 - LEAK) * res_ref[...]
                    + LEAK * jnp.tanh(s + proj_ref[...] + BIAS))


_finish = pl.pallas_call(
    _finish_body,
    out_shape=jax.ShapeDtypeStruct((128, 128), jnp.float32),
)


def kernel(proj_vars, res_state, wr_rows, wr_cols, wr_vals):
    partials = _spmv(res_state, wr_rows, wr_cols, wr_vals)
    out = _finish(partials.reshape(NW, 128, 128),
                  res_state.reshape(128, 128),
                  proj_vars.reshape(128, 128))
    return out.reshape(RES)


# speedup vs baseline: 371.7282x; 1.5853x over previous
import jax
import jax.numpy as jnp
from jax import lax
from jax.experimental import pallas as pl
from jax.experimental.pallas import tpu as pltpu
from jax.experimental.pallas import tpu_sc as plsc

RES = 16384
NNZ = 2684354
LEAK = 0.6
BIAS = 1.6

NC = 2
NS = 16
NW = NC * NS
LANES = 16

CHUNK = 4096
VPC = CHUNK // LANES
NFULL = NNZ // CHUNK
TBASE = NFULL * CHUNK
TAIL = NNZ - TBASE
TFULL = TAIL // LANES
TREM = TAIL - TFULL * LANES
NMAX = (NFULL + NW - 1) // NW


def _spmv_body(res_hbm, rows_hbm, cols_hbm, vals_hbm, out_hbm,
               state_v, acc_v,
               rowbuf0, rowbuf1, colbuf0, colbuf1, valbuf0, valbuf1,
               sem0, sem1):
    wid = lax.axis_index("s") * NC + lax.axis_index("c")
    nmine = (NFULL - wid + NW - 1) // NW
    rowbufs = (rowbuf0, rowbuf1)
    colbufs = (colbuf0, colbuf1)
    valbufs = (valbuf0, valbuf1)
    sems = (sem0, sem1)

    pltpu.sync_copy(res_hbm, state_v)
    zeros = jnp.zeros((LANES,), jnp.float32)

    def _z(i, c):
        acc_v[pl.ds(i * LANES, LANES)] = zeros
        return c
    lax.fori_loop(0, RES // LANES, _z, 0, unroll=8)

    def _start(k, b):
        base = (wid + k * NW) * CHUNK
        pltpu.async_copy(rows_hbm.at[pl.ds(base, CHUNK)], rowbufs[b], sems[b])
        pltpu.async_copy(cols_hbm.at[pl.ds(base, CHUNK)], colbufs[b], sems[b])
        pltpu.async_copy(vals_hbm.at[pl.ds(base, CHUNK)], valbufs[b], sems[b])

    def _drain(k, b):
        base = (wid + k * NW) * CHUNK
        pltpu.make_async_copy(rows_hbm.at[pl.ds(base, CHUNK)], rowbufs[b], sems[b]).wait()
        pltpu.make_async_copy(cols_hbm.at[pl.ds(base, CHUNK)], colbufs[b], sems[b]).wait()
        pltpu.make_async_copy(vals_hbm.at[pl.ds(base, CHUNK)], valbufs[b], sems[b]).wait()

    def _process(b):
        def _v(j, cc):
            off = j * LANES
            idx = colbufs[b][pl.ds(off, LANES)]
            r = rowbufs[b][pl.ds(off, LANES)]
            v = valbufs[b][pl.ds(off, LANES)]
            g = plsc.load_gather(state_v, [idx])
            plsc.addupdate_scatter(acc_v, [r], v * g)
            return cc
        lax.fori_loop(0, VPC, _v, 0, unroll=8)

    _start(0, 0)

    def _outer(t, c):
        for b in range(2):
            k = 2 * t + b

            @pl.when(k + 1 < nmine)
            def _pref():
                _start(k + 1, (b + 1) % 2)

            @pl.when(k < nmine)
            def _proc():
                _drain(k, b)
                _process(b)
        return c
    lax.fori_loop(0, (NMAX + 1) // 2, _outer, 0)

    @pl.when(wid == NW - 1)
    def _tail():
        pltpu.sync_copy(rows_hbm.at[pl.ds(TBASE, TAIL)], rowbuf0.at[pl.ds(0, TAIL)])
        pltpu.sync_copy(cols_hbm.at[pl.ds(TBASE, TAIL)], colbuf0.at[pl.ds(0, TAIL)])
        pltpu.sync_copy(vals_hbm.at[pl.ds(TBASE, TAIL)], valbuf0.at[pl.ds(0, TAIL)])

        def _v(j, cc):
            off = j * LANES
            idx = colbuf0[pl.ds(off, LANES)]
            r = rowbuf0[pl.ds(off, LANES)]
            v = valbuf0[pl.ds(off, LANES)]
            g = plsc.load_gather(state_v, [idx])
            plsc.addupdate_scatter(acc_v, [r], v * g)
            return cc
        lax.fori_loop(0, TFULL, _v, 0)

        off = TFULL * LANES
        m = lax.iota(jnp.int32, LANES) < TREM
        idx = colbuf0[pl.ds(off, LANES)]
        r = rowbuf0[pl.ds(off, LANES)]
        v = valbuf0[pl.ds(off, LANES)]
        g = plsc.load_gather(state_v, [idx], mask=m)
        plsc.addupdate_scatter(acc_v, [r], v * g, mask=m)

    pltpu.sync_copy(acc_v, out_hbm.at[wid])


_spmv = pl.kernel(
    _spmv_body,
    out_type=jax.ShapeDtypeStruct((NW, RES), jnp.float32),
    mesh=plsc.VectorSubcoreMesh(core_axis_name="c", subcore_axis_name="s"),
    compiler_params=pltpu.CompilerParams(needs_layout_passes=False),
    scratch_types=[
        pltpu.VMEM((RES,), jnp.float32),
        pltpu.VMEM((RES,), jnp.float32),
        pltpu.VMEM((CHUNK,), jnp.int32),
        pltpu.VMEM((CHUNK,), jnp.int32),
        pltpu.VMEM((CHUNK,), jnp.int32),
        pltpu.VMEM((CHUNK,), jnp.int32),
        pltpu.VMEM((CHUNK,), jnp.float32),
        pltpu.VMEM((CHUNK,), jnp.float32),
        pltpu.SemaphoreType.DMA,
        pltpu.SemaphoreType.DMA,
    ],
)


def _finish_body(part_ref, res_ref, proj_ref, out_ref):
    s = jnp.sum(part_ref[...], axis=0)
    out_ref[...] = ((1.0 - LEAK) * res_ref[...]
                    + LEAK * jnp.tanh(s + proj_ref[...] + BIAS))


_finish = pl.pallas_call(
    _finish_body,
    out_shape=jax.ShapeDtypeStruct((128, 128), jnp.float32),
)


def kernel(proj_vars, res_state, wr_rows, wr_cols, wr_vals):
    partials = _spmv(res_state, wr_rows, wr_cols, wr_vals)
    out = _finish(partials.reshape(NW, 128, 128),
                  res_state.reshape(128, 128),
                  proj_vars.reshape(128, 128))
    return out.reshape(RES)


# speedup vs baseline: 608.5121x; 1.6370x over previous
import jax
import jax.numpy as jnp
from jax import lax
from jax.experimental import pallas as pl
from jax.experimental.pallas import tpu as pltpu
from jax.experimental.pallas import tpu_sc as plsc

RES = 16384
NNZ = 2684354
LEAK = 0.6
BIAS = 1.6

NC = 2
NS = 16
NW = NC * NS
LANES = 16

CHUNK = 4096
VPC = CHUNK // LANES
NFULL = NNZ // CHUNK
TBASE = NFULL * CHUNK
TAIL = NNZ - TBASE
TFULL = TAIL // LANES
TREM = TAIL - TFULL * LANES
NMAX = (NFULL + NW - 1) // NW


def _spmv_body(res_hbm, rows_hbm, cols_hbm, vals_hbm, out_hbm,
               state_v, acc_v,
               rowbuf0, rowbuf1, colbuf0, colbuf1, valbuf0, valbuf1,
               sem0, sem1):
    wid = lax.axis_index("s") * NC + lax.axis_index("c")
    nmine = (NFULL - wid + NW - 1) // NW
    rowbufs = (rowbuf0, rowbuf1)
    colbufs = (colbuf0, colbuf1)
    valbufs = (valbuf0, valbuf1)
    sems = (sem0, sem1)

    pltpu.sync_copy(res_hbm, state_v)
    zeros = jnp.zeros((LANES,), jnp.float32)

    @plsc.parallel_loop(0, RES // LANES, unroll=8)
    def _z(i):
        acc_v[pl.ds(i * LANES, LANES)] = zeros

    def _start(k, b):
        base = (wid + k * NW) * CHUNK
        pltpu.async_copy(rows_hbm.at[pl.ds(base, CHUNK)], rowbufs[b], sems[b])
        pltpu.async_copy(cols_hbm.at[pl.ds(base, CHUNK)], colbufs[b], sems[b])
        pltpu.async_copy(vals_hbm.at[pl.ds(base, CHUNK)], valbufs[b], sems[b])

    def _drain(k, b):
        base = (wid + k * NW) * CHUNK
        pltpu.make_async_copy(rows_hbm.at[pl.ds(base, CHUNK)], rowbufs[b], sems[b]).wait()
        pltpu.make_async_copy(cols_hbm.at[pl.ds(base, CHUNK)], colbufs[b], sems[b]).wait()
        pltpu.make_async_copy(vals_hbm.at[pl.ds(base, CHUNK)], valbufs[b], sems[b]).wait()

    def _process(b):
        @plsc.parallel_loop(0, VPC, unroll=8)
        def _v(j):
            off = j * LANES
            idx = colbufs[b][pl.ds(off, LANES)]
            r = rowbufs[b][pl.ds(off, LANES)]
            v = valbufs[b][pl.ds(off, LANES)]
            g = plsc.load_gather(state_v, [idx])
            plsc.addupdate_scatter(acc_v, [r], v * g)

    _start(0, 0)

    def _outer(t, c):
        for b in range(2):
            k = 2 * t + b

            @pl.when(k + 1 < nmine)
            def _pref():
                _start(k + 1, (b + 1) % 2)

            @pl.when(k < nmine)
            def _proc():
                _drain(k, b)
                _process(b)
        return c
    lax.fori_loop(0, (NMAX + 1) // 2, _outer, 0)

    @pl.when(wid == NW - 1)
    def _tail():
        pltpu.sync_copy(rows_hbm.at[pl.ds(TBASE, TAIL)], rowbuf0.at[pl.ds(0, TAIL)])
        pltpu.sync_copy(cols_hbm.at[pl.ds(TBASE, TAIL)], colbuf0.at[pl.ds(0, TAIL)])
        pltpu.sync_copy(vals_hbm.at[pl.ds(TBASE, TAIL)], valbuf0.at[pl.ds(0, TAIL)])

        def _v(j, cc):
            off = j * LANES
            idx = colbuf0[pl.ds(off, LANES)]
            r = rowbuf0[pl.ds(off, LANES)]
            v = valbuf0[pl.ds(off, LANES)]
            g = plsc.load_gather(state_v, [idx])
            plsc.addupdate_scatter(acc_v, [r], v * g)
            return cc
        lax.fori_loop(0, TFULL, _v, 0)

        off = TFULL * LANES
        m = lax.iota(jnp.int32, LANES) < TREM
        idx = colbuf0[pl.ds(off, LANES)]
        r = rowbuf0[pl.ds(off, LANES)]
        v = valbuf0[pl.ds(off, LANES)]
        g = plsc.load_gather(state_v, [idx], mask=m)
        plsc.addupdate_scatter(acc_v, [r], v * g, mask=m)

    pltpu.sync_copy(acc_v, out_hbm.at[wid])


_spmv = pl.kernel(
    _spmv_body,
    out_type=jax.ShapeDtypeStruct((NW, RES), jnp.float32),
    mesh=plsc.VectorSubcoreMesh(core_axis_name="c", subcore_axis_name="s"),
    compiler_params=pltpu.CompilerParams(needs_layout_passes=False),
    scratch_types=[
        pltpu.VMEM((RES,), jnp.float32),
        pltpu.VMEM((RES,), jnp.float32),
        pltpu.VMEM((CHUNK,), jnp.int32),
        pltpu.VMEM((CHUNK,), jnp.int32),
        pltpu.VMEM((CHUNK,), jnp.int32),
        pltpu.VMEM((CHUNK,), jnp.int32),
        pltpu.VMEM((CHUNK,), jnp.float32),
        pltpu.VMEM((CHUNK,), jnp.float32),
        pltpu.SemaphoreType.DMA,
        pltpu.SemaphoreType.DMA,
    ],
)


def _finish_body(part_ref, res_ref, proj_ref, out_ref):
    s = jnp.sum(part_ref[...], axis=0)
    out_ref[...] = ((1.0 - LEAK) * res_ref[...]
                    + LEAK * jnp.tanh(s + proj_ref[...] + BIAS))


_finish = pl.pallas_call(
    _finish_body,
    out_shape=jax.ShapeDtypeStruct((128, 128), jnp.float32),
)


def kernel(proj_vars, res_state, wr_rows, wr_cols, wr_vals):
    partials = _spmv(res_state, wr_rows, wr_cols, wr_vals)
    out = _finish(partials.reshape(NW, 128, 128),
                  res_state.reshape(128, 128),
                  proj_vars.reshape(128, 128))
    return out.reshape(RES)


# speedup vs baseline: 621.3665x; 1.0211x over previous
import jax
import jax.numpy as jnp
from jax import lax
from jax.experimental import pallas as pl
from jax.experimental.pallas import tpu as pltpu
from jax.experimental.pallas import tpu_sc as plsc

RES = 16384
NNZ = 2684354
LEAK = 0.6
BIAS = 1.6

NC = 2
NS = 16
NW = NC * NS
LANES = 16

CHUNK = 4096
NBUF = 3
VPC = CHUNK // LANES
NFULL = NNZ // CHUNK
TBASE = NFULL * CHUNK
TAIL = NNZ - TBASE
TFULL = TAIL // LANES
TREM = TAIL - TFULL * LANES
NMAX = (NFULL + NW - 1) // NW
NOUTER = (NMAX + NBUF - 1) // NBUF


def _spmv_body(res_hbm, rows_hbm, cols_hbm, vals_hbm, out_hbm,
               state_v, acc_v,
               rowbuf0, rowbuf1, rowbuf2,
               colbuf0, colbuf1, colbuf2,
               valbuf0, valbuf1, valbuf2,
               sem0, sem1, sem2):
    wid = lax.axis_index("s") * NC + lax.axis_index("c")
    nmine = (NFULL - wid + NW - 1) // NW
    rowbufs = (rowbuf0, rowbuf1, rowbuf2)
    colbufs = (colbuf0, colbuf1, colbuf2)
    valbufs = (valbuf0, valbuf1, valbuf2)
    sems = (sem0, sem1, sem2)

    def _start(k, b):
        base = (wid + k * NW) * CHUNK
        pltpu.async_copy(rows_hbm.at[pl.ds(base, CHUNK)], rowbufs[b], sems[b])
        pltpu.async_copy(cols_hbm.at[pl.ds(base, CHUNK)], colbufs[b], sems[b])
        pltpu.async_copy(vals_hbm.at[pl.ds(base, CHUNK)], valbufs[b], sems[b])

    def _drain(k, b):
        base = (wid + k * NW) * CHUNK
        pltpu.make_async_copy(rows_hbm.at[pl.ds(base, CHUNK)], rowbufs[b], sems[b]).wait()
        pltpu.make_async_copy(cols_hbm.at[pl.ds(base, CHUNK)], colbufs[b], sems[b]).wait()
        pltpu.make_async_copy(vals_hbm.at[pl.ds(base, CHUNK)], valbufs[b], sems[b]).wait()

    _start(0, 0)

    @pl.when(1 < nmine)
    def _p1():
        _start(1, 1)

    pltpu.sync_copy(res_hbm, state_v)
    zeros = jnp.zeros((LANES,), jnp.float32)

    @plsc.parallel_loop(0, RES // LANES, unroll=16)
    def _z(i):
        acc_v[pl.ds(i * LANES, LANES)] = zeros

    def _process(b):
        @plsc.parallel_loop(0, VPC, unroll=16)
        def _v(j):
            off = j * LANES
            idx = colbufs[b][pl.ds(off, LANES)]
            r = rowbufs[b][pl.ds(off, LANES)]
            v = valbufs[b][pl.ds(off, LANES)]
            g = plsc.load_gather(state_v, [idx])
            plsc.addupdate_scatter(acc_v, [r], v * g)

    def _outer(t, c):
        for b in range(NBUF):
            k = NBUF * t + b

            @pl.when(k + 2 < nmine)
            def _pref():
                _start(k + 2, (b + 2) % NBUF)

            @pl.when(k < nmine)
            def _proc():
                _drain(k, b)
                _process(b)
        return c
    lax.fori_loop(0, NOUTER, _outer, 0)

    @pl.when(wid == NW - 1)
    def _tail():
        pltpu.sync_copy(rows_hbm.at[pl.ds(TBASE, TAIL)], rowbuf0.at[pl.ds(0, TAIL)])
        pltpu.sync_copy(cols_hbm.at[pl.ds(TBASE, TAIL)], colbuf0.at[pl.ds(0, TAIL)])
        pltpu.sync_copy(vals_hbm.at[pl.ds(TBASE, TAIL)], valbuf0.at[pl.ds(0, TAIL)])

        @plsc.parallel_loop(0, TFULL, unroll=1)
        def _v(j):
            off = j * LANES
            idx = colbuf0[pl.ds(off, LANES)]
            r = rowbuf0[pl.ds(off, LANES)]
            v = valbuf0[pl.ds(off, LANES)]
            g = plsc.load_gather(state_v, [idx])
            plsc.addupdate_scatter(acc_v, [r], v * g)

        off = TFULL * LANES
        m = lax.iota(jnp.int32, LANES) < TREM
        idx = colbuf0[pl.ds(off, LANES)]
        r = rowbuf0[pl.ds(off, LANES)]
        v = valbuf0[pl.ds(off, LANES)]
        g = plsc.load_gather(state_v, [idx], mask=m)
        plsc.addupdate_scatter(acc_v, [r], v * g, mask=m)

    pltpu.sync_copy(acc_v, out_hbm.at[wid])


_spmv = pl.kernel(
    _spmv_body,
    out_type=jax.ShapeDtypeStruct((NW, RES), jnp.float32),
    mesh=plsc.VectorSubcoreMesh(core_axis_name="c", subcore_axis_name="s"),
    compiler_params=pltpu.CompilerParams(needs_layout_passes=False),
    scratch_types=[
        pltpu.VMEM((RES,), jnp.float32),
        pltpu.VMEM((RES,), jnp.float32),
        pltpu.VMEM((CHUNK,), jnp.int32),
        pltpu.VMEM((CHUNK,), jnp.int32),
        pltpu.VMEM((CHUNK,), jnp.int32),
        pltpu.VMEM((CHUNK,), jnp.int32),
        pltpu.VMEM((CHUNK,), jnp.int32),
        pltpu.VMEM((CHUNK,), jnp.int32),
        pltpu.VMEM((CHUNK,), jnp.float32),
        pltpu.VMEM((CHUNK,), jnp.float32),
        pltpu.VMEM((CHUNK,), jnp.float32),
        pltpu.SemaphoreType.DMA,
        pltpu.SemaphoreType.DMA,
        pltpu.SemaphoreType.DMA,
    ],
)


def _finish_body(part_ref, res_ref, proj_ref, out_ref):
    s = jnp.sum(part_ref[...], axis=0)
    out_ref[...] = ((1.0 - LEAK) * res_ref[...]
                    + LEAK * jnp.tanh(s + proj_ref[...] + BIAS))


_finish = pl.pallas_call(
    _finish_body,
    out_shape=jax.ShapeDtypeStruct((128, 128), jnp.float32),
)


def kernel(proj_vars, res_state, wr_rows, wr_cols, wr_vals):
    partials = _spmv(res_state, wr_rows, wr_cols, wr_vals)
    out = _finish(partials.reshape(NW, 128, 128),
                  res_state.reshape(128, 128),
                  proj_vars.reshape(128, 128))
    return out.reshape(RES)
